# Initial kernel scaffold; baseline (speedup 1.0000x reference)
import jax
import jax.numpy as jnp
from jax.experimental import pallas as pl


def kernel(previous_state, current_state, history_text, current_text, history_roles, current_roles, text_table, state_embedding):
    raise NotImplementedError("write your pallas kernel here")


# speedup vs baseline: 3.5752x; 3.5752x over previous
import math

import jax
import jax.numpy as jnp
import numpy as np
from jax import lax
from jax.experimental import pallas as pl
from jax.experimental.pallas import tpu as pltpu
from jax.experimental.pallas import tpu_sc as plsc

VOCAB = 100000
S_LAB = 35
MAX_CARD = 256
D = 128
B = 1024
HIST = 200
CUR = 50

NC = 2
NS = 16
NW = NC * NS

N_HIST = B * HIST
N_CUR = B * CUR
N_ROWS = N_HIST + N_CUR
ROWS_PER_TILE = N_ROWS // NW
CHUNK = 80
CHUNKS_PER_TILE = ROWS_PER_TILE // CHUNK
N_CHUNKS = N_ROWS // CHUNK


def _positional_encoding(seq_len, d_model):
    pos = np.arange(seq_len, dtype=np.float32)[:, None]
    div = np.exp(np.arange(0, d_model, 2, dtype=np.float32) * (-math.log(10000.0) / d_model))
    pe = np.zeros((seq_len, d_model), dtype=np.float32)
    pe[:, 0::2] = np.sin(pos * div)
    pe[:, 1::2] = np.cos(pos * div)
    return pe


_PE_NP = _positional_encoding(HIST, D)


def _sc_gather_body(tidx_hbm, cidx_hbm, table_hbm, combo_hbm, hist_out, cur_out,
                    tidx_v, cidx_v, rows_t, rows_c, sem_t, sem_c):
    wid = lax.axis_index("s") * NC + lax.axis_index("c")
    pltpu.sync_copy(tidx_hbm.at[wid], tidx_v)
    pltpu.sync_copy(cidx_hbm.at[wid], cidx_v)

    def chunk_body(c, carry):
        ct = pltpu.async_copy(table_hbm.at[tidx_v.at[c]], rows_t, sem_t)
        cc = pltpu.async_copy(combo_hbm.at[cidx_v.at[c]], rows_c, sem_c)
        ct.wait()
        cc.wait()

        def add_row(r, carry2):
            for j in range(D // 16):
                sl = pl.ds(16 * j, 16)
                rows_t[r, sl] = rows_t[r, sl] + rows_c[r, sl]
            return carry2

        lax.fori_loop(0, CHUNK, add_row, 0)

        base = wid * ROWS_PER_TILE + c * CHUNK

        @pl.when(base < N_HIST)
        def _():
            pltpu.sync_copy(rows_t, hist_out.at[pl.ds(base, CHUNK)])

        @pl.when(base >= N_HIST)
        def _():
            pltpu.sync_copy(rows_t, cur_out.at[pl.ds(base - N_HIST, CHUNK)])

        return carry

    lax.fori_loop(0, CHUNKS_PER_TILE, chunk_body, 0)


_sc_gather = pl.kernel(
    _sc_gather_body,
    out_type=[
        jax.ShapeDtypeStruct((N_HIST, D), jnp.float32),
        jax.ShapeDtypeStruct((N_CUR, D), jnp.float32),
    ],
    mesh=plsc.VectorSubcoreMesh(core_axis_name="c", subcore_axis_name="s"),
    scratch_types=[
        pltpu.VMEM((CHUNKS_PER_TILE, CHUNK), jnp.int32),
        pltpu.VMEM((CHUNKS_PER_TILE, CHUNK), jnp.int32),
        pltpu.VMEM((CHUNK, D), jnp.float32),
        pltpu.VMEM((CHUNK, D), jnp.float32),
        pltpu.SemaphoreType.DMA,
        pltpu.SemaphoreType.DMA,
    ],
)


def _state_mm_body(prev_ref, cur_ref, emb_ref, outp_ref, outc_ref):
    for i in range(S_LAB):
        e = emb_ref[i]
        outp_ref[:, i, :] = jnp.dot(prev_ref[:, i, :], e, preferred_element_type=jnp.float32)
        outc_ref[:, i, :] = jnp.dot(cur_ref[:, i, :], e, preferred_element_type=jnp.float32)


_BB = 128

_state_mm = pl.pallas_call(
    _state_mm_body,
    grid=(B // _BB,),
    in_specs=[
        pl.BlockSpec((_BB, S_LAB, MAX_CARD), lambda b: (b, 0, 0)),
        pl.BlockSpec((_BB, S_LAB, MAX_CARD), lambda b: (b, 0, 0)),
        pl.BlockSpec((S_LAB, MAX_CARD, D), lambda b: (0, 0, 0)),
    ],
    out_specs=[
        pl.BlockSpec((_BB, S_LAB, D), lambda b: (b, 0, 0)),
        pl.BlockSpec((_BB, S_LAB, D), lambda b: (b, 0, 0)),
    ],
    out_shape=[
        jax.ShapeDtypeStruct((B, S_LAB, D), jnp.float32),
        jax.ShapeDtypeStruct((B, S_LAB, D), jnp.float32),
    ],
)


def kernel(previous_state, current_state, history_text, current_text,
           history_roles, current_roles, text_table, state_embedding):
    pe = jnp.asarray(_PE_NP)
    t4 = text_table[:4]
    combo_h = (t4[:, None, :] + 2.0 * pe[None, :, :]).reshape(4 * HIST, D)
    combo_c = (t4[:, None, :] + 2.0 * pe[None, :CUR, :]).reshape(4 * CUR, D)
    combo = jnp.concatenate([combo_h, combo_c], axis=0)

    pos_h = jnp.arange(HIST, dtype=jnp.int32)[None, :]
    pos_c = jnp.arange(CUR, dtype=jnp.int32)[None, :]
    cidx = jnp.concatenate([
        (history_roles * HIST + pos_h).reshape(-1),
        (4 * HIST + current_roles * CUR + pos_c).reshape(-1),
    ]).reshape(NW, CHUNKS_PER_TILE, CHUNK)
    tidx = jnp.concatenate([
        history_text.reshape(-1), current_text.reshape(-1)
    ]).reshape(NW, CHUNKS_PER_TILE, CHUNK)

    hist_flat, cur_flat = _sc_gather(tidx, cidx, text_table, combo)
    pre_state_embed, cur_state_embed = _state_mm(previous_state, current_state, state_embedding)

    return (pre_state_embed, cur_state_embed,
            hist_flat.reshape(B, HIST, D), cur_flat.reshape(B, CUR, D))


# speedup vs baseline: 4.1526x; 1.1615x over previous
import math

import jax
import jax.numpy as jnp
import numpy as np
from jax import lax
from jax.experimental import pallas as pl
from jax.experimental.pallas import tpu as pltpu
from jax.experimental.pallas import tpu_sc as plsc

VOCAB = 100000
S_LAB = 35
MAX_CARD = 256
D = 128
B = 1024
HIST = 200
CUR = 50

NC = 2
NS = 16
NW = NC * NS

N_HIST = B * HIST
N_CUR = B * CUR
N_ROWS = N_HIST + N_CUR
RPT = N_ROWS // NW
CHUNK = 80
CPT = RPT // CHUNK
HIST_TILES = N_HIST // RPT
HIST_REM = N_HIST - HIST_TILES * RPT
COMBO_ROWS = 4 * HIST + 4 * CUR


def _positional_encoding(seq_len, d_model):
    pos = np.arange(seq_len, dtype=np.float32)[:, None]
    div = np.exp(np.arange(0, d_model, 2, dtype=np.float32) * (-math.log(10000.0) / d_model))
    pe = np.zeros((seq_len, d_model), dtype=np.float32)
    pe[:, 0::2] = np.sin(pos * div)
    pe[:, 1::2] = np.cos(pos * div)
    return pe


_PE_NP = _positional_encoding(HIST, D)


def _sc_gather_body(ht_hbm, ct_hbm, hr_hbm, cr_hbm, table_hbm, combo_hbm,
                    hist_out, cur_out,
                    tv, rv, rt0, rc0, rt1, rc1, st0, st1,
                    sem_t0, sem_c0, sem_t1, sem_c1, sem_s0, sem_s1):
    wid = lax.axis_index("s") * NC + lax.axis_index("c")
    row0 = wid * RPT

    @pl.when(wid < HIST_TILES)
    def _():
        pltpu.sync_copy(ht_hbm.at[pl.ds(row0, RPT)], tv)
        pltpu.sync_copy(hr_hbm.at[pl.ds(row0, RPT)], rv)

    @pl.when(wid == HIST_TILES)
    def _():
        pltpu.sync_copy(ht_hbm.at[pl.ds(HIST_TILES * RPT, HIST_REM)], tv.at[pl.ds(0, HIST_REM)])
        pltpu.sync_copy(ct_hbm.at[pl.ds(0, RPT - HIST_REM)], tv.at[pl.ds(HIST_REM, RPT - HIST_REM)])
        pltpu.sync_copy(hr_hbm.at[pl.ds(HIST_TILES * RPT, HIST_REM)], rv.at[pl.ds(0, HIST_REM)])
        pltpu.sync_copy(cr_hbm.at[pl.ds(0, RPT - HIST_REM)], rv.at[pl.ds(HIST_REM, RPT - HIST_REM)])

    @pl.when(wid > HIST_TILES)
    def _():
        pltpu.sync_copy(ct_hbm.at[pl.ds(row0 - N_HIST, RPT)], tv)
        pltpu.sync_copy(cr_hbm.at[pl.ds(row0 - N_HIST, RPT)], rv)

    iota16 = lax.broadcasted_iota(jnp.int32, (16,), 0)

    def cidx_body(i, carry):
        sl = pl.ds(16 * i, 16)
        r = rv[sl]
        n = row0 + 16 * i + iota16
        is_hist = n < N_HIST
        ch = r * HIST + lax.rem(n, HIST)
        cc = 4 * HIST + r * CUR + lax.rem(n - N_HIST, CUR)
        rv[sl] = jnp.where(is_hist, ch, cc)
        return carry

    lax.fori_loop(0, RPT // 16, cidx_body, 0)

    def fire_gathers(c, rt, rc, sem_t, sem_c):
        tsl = tv.at[pl.ds(c * CHUNK, CHUNK)]
        csl = rv.at[pl.ds(c * CHUNK, CHUNK)]
        pltpu.async_copy(table_hbm.at[tsl], rt, sem_t)
        pltpu.async_copy(combo_hbm.at[csl], rc, sem_c)

    def wait_gathers(rt, rc, sem_t, sem_c):
        pltpu.make_async_copy(table_hbm.at[tv.at[pl.ds(0, CHUNK)]], rt, sem_t).wait()
        pltpu.make_async_copy(combo_hbm.at[rv.at[pl.ds(0, CHUNK)]], rc, sem_c).wait()

    def fire_store(c, st, sem_s):
        base = row0 + c * CHUNK

        @pl.when(base < N_HIST)
        def _():
            pltpu.async_copy(st, hist_out.at[pl.ds(base, CHUNK)], sem_s)

        @pl.when(base >= N_HIST)
        def _():
            pltpu.async_copy(st, cur_out.at[pl.ds(base - N_HIST, CHUNK)], sem_s)

    def wait_store(st, sem_s):
        pltpu.make_async_copy(st, hist_out.at[pl.ds(0, CHUNK)], sem_s).wait()

    def add_rows(rt, rc, st):
        def add_row(r, carry):
            for j in range(D // 16):
                sl = pl.ds(16 * j, 16)
                st[r, sl] = rt[r, sl] + rc[r, sl]
            return carry

        lax.fori_loop(0, CHUNK, add_row, 0)

    fire_gathers(0, rt0, rc0, sem_t0, sem_c0)
    fire_gathers(1, rt1, rc1, sem_t1, sem_c1)

    def pair_body(k, carry):
        c = 2 * k
        wait_gathers(rt0, rc0, sem_t0, sem_c0)

        @pl.when(k > 0)
        def _():
            wait_store(st0, sem_s0)

        add_rows(rt0, rc0, st0)

        @pl.when(c + 2 < CPT)
        def _():
            fire_gathers(c + 2, rt0, rc0, sem_t0, sem_c0)

        fire_store(c, st0, sem_s0)

        wait_gathers(rt1, rc1, sem_t1, sem_c1)

        @pl.when(k > 0)
        def _():
            wait_store(st1, sem_s1)

        add_rows(rt1, rc1, st1)

        @pl.when(c + 3 < CPT)
        def _():
            fire_gathers(c + 3, rt1, rc1, sem_t1, sem_c1)

        fire_store(c + 1, st1, sem_s1)
        return carry

    lax.fori_loop(0, CPT // 2, pair_body, 0)
    wait_store(st0, sem_s0)
    wait_store(st1, sem_s1)


_sc_gather = pl.kernel(
    _sc_gather_body,
    out_type=[
        jax.ShapeDtypeStruct((N_HIST, D), jnp.float32),
        jax.ShapeDtypeStruct((N_CUR, D), jnp.float32),
    ],
    mesh=plsc.VectorSubcoreMesh(core_axis_name="c", subcore_axis_name="s"),
    scratch_types=[
        pltpu.VMEM((RPT,), jnp.int32),
        pltpu.VMEM((RPT,), jnp.int32),
        pltpu.VMEM((CHUNK, D), jnp.float32),
        pltpu.VMEM((CHUNK, D), jnp.float32),
        pltpu.VMEM((CHUNK, D), jnp.float32),
        pltpu.VMEM((CHUNK, D), jnp.float32),
        pltpu.VMEM((CHUNK, D), jnp.float32),
        pltpu.VMEM((CHUNK, D), jnp.float32),
        pltpu.SemaphoreType.DMA,
        pltpu.SemaphoreType.DMA,
        pltpu.SemaphoreType.DMA,
        pltpu.SemaphoreType.DMA,
        pltpu.SemaphoreType.DMA,
        pltpu.SemaphoreType.DMA,
    ],
)


def _state_mm_body(prev_ref, cur_ref, emb_ref, outp_ref, outc_ref):
    for i in range(S_LAB):
        e = emb_ref[pl.ds(i * MAX_CARD, MAX_CARD), :]
        outp_ref[:, pl.ds(i * D, D)] = jnp.dot(
            prev_ref[:, pl.ds(i * MAX_CARD, MAX_CARD)], e,
            preferred_element_type=jnp.float32)
        outc_ref[:, pl.ds(i * D, D)] = jnp.dot(
            cur_ref[:, pl.ds(i * MAX_CARD, MAX_CARD)], e,
            preferred_element_type=jnp.float32)


_BB = 128

_state_mm = pl.pallas_call(
    _state_mm_body,
    grid=(B // _BB,),
    in_specs=[
        pl.BlockSpec((_BB, S_LAB * MAX_CARD), lambda b: (b, 0)),
        pl.BlockSpec((_BB, S_LAB * MAX_CARD), lambda b: (b, 0)),
        pl.BlockSpec((S_LAB * MAX_CARD, D), lambda b: (0, 0)),
    ],
    out_specs=[
        pl.BlockSpec((_BB, S_LAB * D), lambda b: (b, 0)),
        pl.BlockSpec((_BB, S_LAB * D), lambda b: (b, 0)),
    ],
    out_shape=[
        jax.ShapeDtypeStruct((B, S_LAB * D), jnp.float32),
        jax.ShapeDtypeStruct((B, S_LAB * D), jnp.float32),
    ],
)


def kernel(previous_state, current_state, history_text, current_text,
           history_roles, current_roles, text_table, state_embedding):
    pe = jnp.asarray(_PE_NP)
    t4 = text_table[:4]
    combo_h = (t4[:, None, :] + 2.0 * pe[None, :, :]).reshape(4 * HIST, D)
    combo_c = (t4[:, None, :] + 2.0 * pe[None, :CUR, :]).reshape(4 * CUR, D)
    combo = jnp.concatenate([combo_h, combo_c], axis=0)

    hist_flat, cur_flat = _sc_gather(
        history_text.reshape(-1), current_text.reshape(-1),
        history_roles.reshape(-1), current_roles.reshape(-1),
        text_table, combo)

    pre2, cur2 = _state_mm(
        previous_state.reshape(B, S_LAB * MAX_CARD),
        current_state.reshape(B, S_LAB * MAX_CARD),
        state_embedding.reshape(S_LAB * MAX_CARD, D))

    return (pre2.reshape(B, S_LAB, D), cur2.reshape(B, S_LAB, D),
            hist_flat.reshape(B, HIST, D), cur_flat.reshape(B, CUR, D))
